# baseline (speedup 1.0000x reference)
import functools

import jax
import jax.numpy as jnp
from jax import lax
from jax.experimental import pallas as pl
from jax.experimental.pallas import tpu as pltpu
from jax.experimental.pallas import tpu_sc as plsc

D = 64
SCALE = 8.0
IW = 128
NC = 2
NS = 16
NW = NC * NS
K = 8
CHUNK = K * IW


@functools.partial(jax.jit, static_argnames=("n_rows",))
def _emb_lookup(t128, idx2d, *, n_rows):
    rows_per_w = n_rows // NW
    n_chunks = rows_per_w // K
    n_idx = n_rows * IW

    mesh = plsc.VectorSubcoreMesh(core_axis_name="c", subcore_axis_name="s")

    @functools.partial(
        pl.kernel,
        mesh=mesh,
        out_type=jax.ShapeDtypeStruct((n_idx, D), jnp.float32),
        scratch_types=[
            pltpu.VMEM((K, IW), jnp.int32),
            pltpu.VMEM((K, IW), jnp.int32),
            pltpu.VMEM((2, IW, IW), jnp.float32),
            pltpu.VMEM((CHUNK // 2, D), jnp.float32),
            pltpu.SemaphoreType.DMA,
            pltpu.SemaphoreType.DMA,
        ],
    )
    def body(t_hbm, idx_hbm, out_hbm, idx_v, jdx_v, pair_v, out_v, semA, semB):
        wid = lax.axis_index("s") * NC + lax.axis_index("c")
        row0 = wid * rows_per_w
        sems = (semA, semB)

        def select_group(g, buf):
            obase = (g % (K // 2)) * IW

            def sel_block(blk, carry):
                hv = idx_v[g, pl.ds(blk * 16, 16)] & 1
                for t in range(16):
                    r = blk * 16 + t
                    p = hv[t] == 0
                    for s in range(D // 16):
                        a = pair_v[buf, r, pl.ds(s * 16, 16)]
                        b = pair_v[buf, r, pl.ds(D + s * 16, 16)]
                        out_v[obase + r, pl.ds(s * 16, 16)] = (
                            jnp.where(p, a, b) * SCALE
                        )
                return carry

            lax.fori_loop(0, IW // 16, sel_block, 0)

        def chunk_body(c, carry):
            r0 = row0 + c * K
            pltpu.sync_copy(idx_hbm.at[pl.ds(r0, K)], idx_v)
            for r in range(K):
                for s in range(IW // 16):
                    jdx_v[r, pl.ds(s * 16, 16)] = (
                        idx_v[r, pl.ds(s * 16, 16)] >> 1
                    )
            for half in range(2):
                g0 = half * (K // 2)
                gs = list(range(g0, g0 + K // 2))
                cps = {}
                cps[gs[0]] = pltpu.async_copy(
                    t_hbm.at[jdx_v.at[gs[0]]], pair_v.at[gs[0] % 2],
                    sems[gs[0] % 2],
                )
                for g in gs[1:]:
                    cps[g] = pltpu.async_copy(
                        t_hbm.at[jdx_v.at[g]], pair_v.at[g % 2], sems[g % 2]
                    )
                    cps[g - 1].wait()
                    select_group(g - 1, (g - 1) % 2)
                cps[gs[-1]].wait()
                select_group(gs[-1], gs[-1] % 2)
                pltpu.sync_copy(
                    out_v,
                    out_hbm.at[pl.ds(r0 * IW + half * (CHUNK // 2),
                                     CHUNK // 2)],
                )
            return carry

        lax.fori_loop(0, n_chunks, chunk_body, 0)

    return body(t128, idx2d)


def kernel(x, table):
    b, s = x.shape
    n_idx = b * s
    n_rows = n_idx // IW
    idx2d = x.astype(jnp.int32).reshape(n_rows, IW)
    t128 = table.reshape(table.shape[0] // 2, 2 * D)
    out = _emb_lookup(t128, idx2d, n_rows=n_rows)
    return out.reshape(b, s, D)
